# speedup vs baseline: 13.2046x; 13.2046x over previous
import functools

import jax
import jax.numpy as jnp
from jax import lax
from jax.experimental import pallas as pl
from jax.experimental.pallas import tpu as pltpu
from jax.experimental.pallas import tpu_sc as plsc

VOCAB = 100000
VEC_DIM = 128
HIDDEN = 256
LABELS = 16
BATCH = 4096
SEQ = 200

NC = 2
NS = 16
NW = NC * NS
B_PER_W = BATCH // NW
HALF = SEQ // 2
ROWS_PER_W = B_PER_W * 2


def _sc_gather_sum(embed, xr):
  mesh = plsc.VectorSubcoreMesh(core_axis_name="c", subcore_axis_name="s")

  @functools.partial(
      pl.kernel,
      mesh=mesh,
      out_type=jax.ShapeDtypeStruct((BATCH, VEC_DIM), jnp.float32),
      scratch_types=[
          pltpu.VMEM((ROWS_PER_W, HALF), jnp.int32),
          pltpu.VMEM((SEQ, VEC_DIM), jnp.float32),
          pltpu.VMEM((SEQ, VEC_DIM), jnp.float32),
          pltpu.VMEM((B_PER_W, VEC_DIM), jnp.float32),
          pltpu.SemaphoreType.DMA,
          pltpu.SemaphoreType.DMA,
      ],
  )
  def k(embed_hbm, xr_hbm, out_hbm, idx_v, buf0, buf1, m_local, sem0, sem1):
    wid = lax.axis_index("s") * NC + lax.axis_index("c")
    pltpu.sync_copy(xr_hbm.at[pl.ds(wid * ROWS_PER_W, ROWS_PER_W)], idx_v)

    def gather_elem(b, buf, sem):
      r = 2 * b
      pltpu.make_async_copy(
          embed_hbm.at[idx_v.at[r]], buf.at[pl.ds(0, HALF)], sem).start()
      pltpu.make_async_copy(
          embed_hbm.at[idx_v.at[r + 1]], buf.at[pl.ds(HALF, HALF)], sem).start()

    def wait_elem(buf, sem):
      pltpu.make_async_copy(
          embed_hbm.at[idx_v.at[0]], buf.at[pl.ds(0, HALF)], sem).wait()
      pltpu.make_async_copy(
          embed_hbm.at[idx_v.at[0]], buf.at[pl.ds(HALF, HALF)], sem).wait()

    def reduce_elem(buf, b):
      def body(j2, accs):
        j = 2 * j2
        accs = tuple(
            accs[d] + buf[j, d * 16:(d + 1) * 16] for d in range(8))
        accs = tuple(
            accs[d] + buf[j + 1, d * 16:(d + 1) * 16] for d in range(8))
        return accs
      accs = tuple(jnp.zeros((16,), jnp.float32) for _ in range(8))
      accs = lax.fori_loop(0, SEQ // 2, body, accs)
      for d in range(8):
        m_local[b, d * 16:(d + 1) * 16] = accs[d]

    gather_elem(0, buf0, sem0)

    def outer(b2, carry):
      b = 2 * b2
      gather_elem(b + 1, buf1, sem1)
      wait_elem(buf0, sem0)
      reduce_elem(buf0, b)

      @pl.when(b2 < B_PER_W // 2 - 1)
      def _():
        gather_elem(b + 2, buf0, sem0)

      wait_elem(buf1, sem1)
      reduce_elem(buf1, b + 1)
      return carry

    lax.fori_loop(0, B_PER_W // 2, outer, 0)
    pltpu.sync_copy(m_local, out_hbm.at[pl.ds(wid * B_PER_W, B_PER_W)])

  return k(embed, xr)


def _mlp_kernel(msum_ref, w1_ref, b1_ref, g_ref, be_ref, w2_ref, b2_ref,
                out_ref):
  m = msum_ref[...] * (1.0 / SEQ)
  h = lax.dot_general(m, w1_ref[...], (((1,), (1,)), ((), ())),
                      preferred_element_type=jnp.float32)
  h = h + b1_ref[...]
  mu = jnp.mean(h, axis=0, keepdims=True)
  d = h - mu
  var = jnp.mean(d * d, axis=0, keepdims=True)
  hn = d * lax.rsqrt(var + 1e-5) * g_ref[...] + be_ref[...]
  a = jnp.maximum(hn, 0.0)
  out_ref[...] = lax.dot_general(a, w2_ref[...], (((1,), (1,)), ((), ())),
                                 preferred_element_type=jnp.float32) + b2_ref[...]


def kernel(X, embed, W1, b1, gamma, beta, W2, b2):
  xr = X.astype(jnp.int32).reshape(BATCH * 2, HALF)
  msum = _sc_gather_sum(embed, xr)
  out = pl.pallas_call(
      _mlp_kernel,
      out_shape=jax.ShapeDtypeStruct((BATCH, LABELS), jnp.float32),
  )(msum, W1, b1.reshape(1, HIDDEN), gamma.reshape(1, HIDDEN),
    beta.reshape(1, HIDDEN), W2, b2.reshape(1, LABELS))
  return out


# speedup vs baseline: 15.8907x; 1.2034x over previous
import functools

import jax
import jax.numpy as jnp
from jax import lax
from jax.experimental import pallas as pl
from jax.experimental.pallas import tpu as pltpu
from jax.experimental.pallas import tpu_sc as plsc

VOCAB = 100000
VEC_DIM = 128
HIDDEN = 256
LABELS = 16
BATCH = 4096
SEQ = 200

NC = 2
NS = 16
NW = NC * NS
B_PER_W = BATCH // NW
HALF = SEQ // 2
ROWS_PER_W = B_PER_W * 2


def _sc_gather_sum(embed, xr):
  mesh = plsc.VectorSubcoreMesh(core_axis_name="c", subcore_axis_name="s")

  @functools.partial(
      pl.kernel,
      mesh=mesh,
      out_type=jax.ShapeDtypeStruct((BATCH, VEC_DIM), jnp.float32),
      scratch_types=[
          pltpu.VMEM((ROWS_PER_W, HALF), jnp.int32),
          pltpu.VMEM((HALF, VEC_DIM), jnp.float32),
          pltpu.VMEM((HALF, VEC_DIM), jnp.float32),
          pltpu.VMEM((HALF, VEC_DIM), jnp.float32),
          pltpu.VMEM((HALF, VEC_DIM), jnp.float32),
          pltpu.VMEM((B_PER_W, VEC_DIM), jnp.float32),
          pltpu.SemaphoreType.DMA,
          pltpu.SemaphoreType.DMA,
          pltpu.SemaphoreType.DMA,
          pltpu.SemaphoreType.DMA,
      ],
  )
  def k(embed_hbm, xr_hbm, out_hbm, idx_v, b0, b1, b2, b3, m_local,
        s0, s1, s2, s3):
    bufs = (b0, b1, b2, b3)
    sems = (s0, s1, s2, s3)
    wid = lax.axis_index("s") * NC + lax.axis_index("c")
    pltpu.sync_copy(xr_hbm.at[pl.ds(wid * ROWS_PER_W, ROWS_PER_W)], idx_v)

    def gather_half(r, slot):
      pltpu.make_async_copy(
          embed_hbm.at[idx_v.at[r]], bufs[slot], sems[slot]).start()

    def wait_half(slot):
      pltpu.make_async_copy(
          embed_hbm.at[idx_v.at[0]], bufs[slot], sems[slot]).wait()

    def reduce_half(buf, accs):
      def body(j4, accs):
        for u in range(4):
          j = 4 * j4 + u
          accs = tuple(
              accs[d] + buf[j, d * 16:(d + 1) * 16] for d in range(8))
        return accs
      return lax.fori_loop(0, HALF // 4, body, accs)

    for slot in range(3):
      gather_half(slot, slot)

    def outer(g, carry):
      r0 = 4 * g
      accs = None
      for s in range(4):
        @pl.when(r0 + s + 3 < ROWS_PER_W)
        def _(s=s):
          gather_half(r0 + s + 3, (s + 3) % 4)

        wait_half(s)
        if s % 2 == 0:
          accs = tuple(jnp.zeros((16,), jnp.float32) for _ in range(8))
        accs = reduce_half(bufs[s], accs)
        if s % 2 == 1:
          b = 2 * g + s // 2
          for d in range(8):
            m_local[b, d * 16:(d + 1) * 16] = accs[d]
      return carry

    lax.fori_loop(0, ROWS_PER_W // 4, outer, 0)
    pltpu.sync_copy(m_local, out_hbm.at[pl.ds(wid * B_PER_W, B_PER_W)])

  return k(embed, xr)


def _mlp_kernel(msum_ref, w1_ref, b1_ref, g_ref, be_ref, w2_ref, b2_ref,
                out_ref):
  m = msum_ref[...] * (1.0 / SEQ)
  h = lax.dot_general(m, w1_ref[...], (((1,), (1,)), ((), ())),
                      preferred_element_type=jnp.float32)
  h = h + b1_ref[...]
  mu = jnp.mean(h, axis=0, keepdims=True)
  d = h - mu
  var = jnp.mean(d * d, axis=0, keepdims=True)
  hn = d * lax.rsqrt(var + 1e-5) * g_ref[...] + be_ref[...]
  a = jnp.maximum(hn, 0.0)
  out_ref[...] = lax.dot_general(a, w2_ref[...], (((1,), (1,)), ((), ())),
                                 preferred_element_type=jnp.float32) + b2_ref[...]


def kernel(X, embed, W1, b1, gamma, beta, W2, b2):
  xr = X.astype(jnp.int32).reshape(BATCH * 2, HALF)
  msum = _sc_gather_sum(embed, xr)
  out = pl.pallas_call(
      _mlp_kernel,
      out_shape=jax.ShapeDtypeStruct((BATCH, LABELS), jnp.float32),
  )(msum, W1, b1.reshape(1, HIDDEN), gamma.reshape(1, HIDDEN),
    beta.reshape(1, HIDDEN), W2, b2.reshape(1, LABELS))
  return out


# speedup vs baseline: 15.9213x; 1.0019x over previous
import functools

import jax
import jax.numpy as jnp
from jax import lax
from jax.experimental import pallas as pl
from jax.experimental.pallas import tpu as pltpu
from jax.experimental.pallas import tpu_sc as plsc

VOCAB = 100000
VEC_DIM = 128
HIDDEN = 256
LABELS = 16
BATCH = 4096
SEQ = 200

NC = 2
NS = 16
NW = NC * NS
B_PER_W = BATCH // NW
HALF = SEQ // 2
ROWS_PER_W = B_PER_W * 2


def _sc_gather_sum(embed, xr):
  mesh = plsc.VectorSubcoreMesh(core_axis_name="c", subcore_axis_name="s")

  @functools.partial(
      pl.kernel,
      mesh=mesh,
      out_type=jax.ShapeDtypeStruct((BATCH, VEC_DIM), jnp.float32),
      scratch_types=[
          pltpu.VMEM((ROWS_PER_W, HALF), jnp.int32),
          pltpu.VMEM((HALF, VEC_DIM), jnp.float32),
          pltpu.VMEM((HALF, VEC_DIM), jnp.float32),
          pltpu.VMEM((HALF, VEC_DIM), jnp.float32),
          pltpu.VMEM((HALF, VEC_DIM), jnp.float32),
          pltpu.VMEM((HALF, VEC_DIM), jnp.float32),
          pltpu.VMEM((HALF, VEC_DIM), jnp.float32),
          pltpu.VMEM((B_PER_W, VEC_DIM), jnp.float32),
          pltpu.SemaphoreType.DMA,
          pltpu.SemaphoreType.DMA,
          pltpu.SemaphoreType.DMA,
          pltpu.SemaphoreType.DMA,
          pltpu.SemaphoreType.DMA,
          pltpu.SemaphoreType.DMA,
      ],
  )
  def k(embed_hbm, xr_hbm, out_hbm, idx_v, b0, b1, b2, b3, b4, b5, m_local,
        s0, s1, s2, s3, s4, s5):
    bufs = (b0, b1, b2, b3, b4, b5)
    sems = (s0, s1, s2, s3, s4, s5)
    wid = lax.axis_index("s") * NC + lax.axis_index("c")
    pltpu.sync_copy(xr_hbm.at[pl.ds(wid * ROWS_PER_W, ROWS_PER_W)], idx_v)

    def gather_half(r, slot):
      pltpu.make_async_copy(
          embed_hbm.at[idx_v.at[r]], bufs[slot], sems[slot]).start()

    def wait_half(slot):
      pltpu.make_async_copy(
          embed_hbm.at[idx_v.at[0]], bufs[slot], sems[slot]).wait()

    def reduce_half(buf, accs):
      def body(j4, accs):
        for u in range(4):
          j = 4 * j4 + u
          accs = tuple(
              accs[d] + buf[j, d * 16:(d + 1) * 16] for d in range(8))
        return accs
      return lax.fori_loop(0, HALF // 4, body, accs)

    for slot in range(5):
      gather_half(slot, slot)

    def outer(g, carry):
      r0 = 6 * g
      accs = None
      for s in range(6):
        @pl.when(r0 + s + 5 < ROWS_PER_W)
        def _(s=s):
          gather_half(r0 + s + 5, (s + 5) % 6)

        wait_half(s)
        if s % 2 == 0:
          accs = tuple(jnp.zeros((16,), jnp.float32) for _ in range(8))
        accs = reduce_half(bufs[s], accs)
        if s % 2 == 1:
          b = 3 * g + s // 2
          for d in range(8):
            m_local[b, d * 16:(d + 1) * 16] = accs[d]
      return carry

    n_full = ROWS_PER_W // 6
    lax.fori_loop(0, n_full, outer, 0)
    accs = None
    for s in range(ROWS_PER_W % 6):
      wait_half(s)
      if s % 2 == 0:
        accs = tuple(jnp.zeros((16,), jnp.float32) for _ in range(8))
      accs = reduce_half(bufs[s], accs)
      if s % 2 == 1:
        b = 3 * n_full + s // 2
        for d in range(8):
          m_local[b, d * 16:(d + 1) * 16] = accs[d]
    pltpu.sync_copy(m_local, out_hbm.at[pl.ds(wid * B_PER_W, B_PER_W)])

  return k(embed, xr)


def _mlp_kernel(msum_ref, w1_ref, b1_ref, g_ref, be_ref, w2_ref, b2_ref,
                out_ref):
  m = msum_ref[...] * (1.0 / SEQ)
  h = lax.dot_general(m, w1_ref[...], (((1,), (1,)), ((), ())),
                      preferred_element_type=jnp.float32)
  h = h + b1_ref[...]
  mu = jnp.mean(h, axis=0, keepdims=True)
  d = h - mu
  var = jnp.mean(d * d, axis=0, keepdims=True)
  hn = d * lax.rsqrt(var + 1e-5) * g_ref[...] + be_ref[...]
  a = jnp.maximum(hn, 0.0)
  out_ref[...] = lax.dot_general(a, w2_ref[...], (((1,), (1,)), ((), ())),
                                 preferred_element_type=jnp.float32) + b2_ref[...]


def kernel(X, embed, W1, b1, gamma, beta, W2, b2):
  xr = X.astype(jnp.int32).reshape(BATCH * 2, HALF)
  msum = _sc_gather_sum(embed, xr)
  out = pl.pallas_call(
      _mlp_kernel,
      out_shape=jax.ShapeDtypeStruct((BATCH, LABELS), jnp.float32),
  )(msum, W1, b1.reshape(1, HIDDEN), gamma.reshape(1, HIDDEN),
    beta.reshape(1, HIDDEN), W2, b2.reshape(1, LABELS))
  return out
